# baseline (speedup 1.0000x reference)
import functools

import jax
import jax.numpy as jnp
from jax import lax
from jax.experimental import pallas as pl
from jax.experimental.pallas import tpu as pltpu
from jax.experimental.pallas import tpu_sc as plsc

NX = 56
NTHETA = 8
PLANE_IN = NX * NX
PLANE_OUT = 4 * PLANE_IN
BC = 4 * 192
NPLANES = BC * NTHETA
NWORKERS = 32
PLANES_PER_W = NPLANES // NWORKERS


def _unpool_body(x_hbm, out_hbm, in_buf, out_buf):
    c = lax.axis_index("c")
    s = lax.axis_index("s")
    wid = s * 2 + c
    pat = lax.iota(jnp.int32, 16) // 2

    def plane_body(i, carry):
        p = wid * PLANES_PER_W + i
        pltpu.sync_copy(x_hbm.at[pl.ds(p * PLANE_IN, PLANE_IN)], in_buf)

        def row_body(r, carry2):
            rb_in = r * NX
            rb_out = r * (4 * NX)
            for o in range(7):
                idx = rb_in + 8 * o + pat
                v = plsc.load_gather(in_buf, [idx])
                out_buf[pl.ds(rb_out + 16 * o, 16)] = v
                out_buf[pl.ds(rb_out + 2 * NX + 16 * o, 16)] = v
            return carry2

        lax.fori_loop(0, NX, row_body, 0)
        pltpu.sync_copy(out_buf, out_hbm.at[pl.ds(p * PLANE_OUT, PLANE_OUT)])
        return carry

    lax.fori_loop(0, PLANES_PER_W, plane_body, 0)


@jax.jit
def kernel(x):
    xf = x.reshape(-1)
    mesh = plsc.VectorSubcoreMesh(core_axis_name="c", subcore_axis_name="s")
    run = pl.kernel(
        _unpool_body,
        out_type=jax.ShapeDtypeStruct((NPLANES * PLANE_OUT,), jnp.float32),
        mesh=mesh,
        scratch_types=[
            pltpu.VMEM((PLANE_IN,), jnp.float32),
            pltpu.VMEM((PLANE_OUT,), jnp.float32),
        ],
        compiler_params=pltpu.CompilerParams(needs_layout_passes=False),
    )
    out = run(xf)
    return out.reshape(4, 192, NTHETA * PLANE_OUT)


# speedup vs baseline: 1.4318x; 1.4318x over previous
import jax
import jax.numpy as jnp
from jax import lax
from jax.experimental import pallas as pl
from jax.experimental.pallas import tpu as pltpu
from jax.experimental.pallas import tpu_sc as plsc

NX = 56
NTHETA = 8
PLANE_IN = NX * NX
PLANE_OUT = 4 * PLANE_IN
BC = 4 * 192
NPLANES = BC * NTHETA
NWORKERS = 32
PLANES_PER_W = NPLANES // NWORKERS


def _unpool_body(x_hbm, out_hbm, in0, in1, ou0, ou1, si0, si1, so0, so1):
    ins = [in0, in1]
    outs = [ou0, ou1]
    sis = [si0, si1]
    sos = [so0, so1]
    c = lax.axis_index("c")
    s = lax.axis_index("s")
    wid = s * 2 + c
    base = wid * PLANES_PER_W
    pat = lax.shift_right_logical(lax.iota(jnp.int32, 16), 1)

    for b in range(2):
        pltpu.async_copy(
            x_hbm.at[pl.ds((base + b) * PLANE_IN, PLANE_IN)], ins[b], sis[b])

    def step(g, carry):
        for b in range(2):
            p = base + g + b
            pltpu.make_async_copy(
                x_hbm.at[pl.ds(p * PLANE_IN, PLANE_IN)], ins[b], sis[b]).wait()

            @pl.when(g > 0)
            def _wait_out():
                pltpu.make_async_copy(
                    outs[b],
                    out_hbm.at[pl.ds((p - 2) * PLANE_OUT, PLANE_OUT)],
                    sos[b]).wait()

            def row_body(r, c2):
                rb_in = r * NX
                rb_out = r * (4 * NX)
                for o in range(7):
                    idx = rb_in + 8 * o + pat
                    v = plsc.load_gather(ins[b], [idx])
                    outs[b][pl.ds(rb_out + 16 * o, 16)] = v
                    outs[b][pl.ds(rb_out + 2 * NX + 16 * o, 16)] = v
                return c2

            lax.fori_loop(0, NX, row_body, 0, unroll=4)

            pltpu.async_copy(
                outs[b], out_hbm.at[pl.ds(p * PLANE_OUT, PLANE_OUT)], sos[b])

            @pl.when(g + 2 < PLANES_PER_W)
            def _start_next_in():
                pltpu.async_copy(
                    x_hbm.at[pl.ds((p + 2) * PLANE_IN, PLANE_IN)],
                    ins[b], sis[b])
        return carry

    lax.fori_loop(0, PLANES_PER_W // 2, lambda i, c2: step(i * 2, c2), 0)

    for b in range(2):
        pltpu.make_async_copy(
            outs[b],
            out_hbm.at[pl.ds((base + PLANES_PER_W - 2 + b) * PLANE_OUT,
                             PLANE_OUT)],
            sos[b]).wait()


@jax.jit
def kernel(x):
    xf = x.reshape(-1)
    mesh = plsc.VectorSubcoreMesh(core_axis_name="c", subcore_axis_name="s")
    run = pl.kernel(
        _unpool_body,
        out_type=jax.ShapeDtypeStruct((NPLANES * PLANE_OUT,), jnp.float32),
        mesh=mesh,
        scratch_types=[
            pltpu.VMEM((PLANE_IN,), jnp.float32),
            pltpu.VMEM((PLANE_IN,), jnp.float32),
            pltpu.VMEM((PLANE_OUT,), jnp.float32),
            pltpu.VMEM((PLANE_OUT,), jnp.float32),
            pltpu.SemaphoreType.DMA,
            pltpu.SemaphoreType.DMA,
            pltpu.SemaphoreType.DMA,
            pltpu.SemaphoreType.DMA,
        ],
        compiler_params=pltpu.CompilerParams(needs_layout_passes=False),
    )
    out = run(xf)
    return out.reshape(4, 192, NTHETA * PLANE_OUT)
